# Initial kernel scaffold; baseline (speedup 1.0000x reference)
import jax
import jax.numpy as jnp
from jax.experimental import pallas as pl


def kernel(input, lang, table_0, table_1):
    raise NotImplementedError("write your pallas kernel here")


# speedup vs baseline: 4.0300x; 4.0300x over previous
import functools

import jax
import jax.numpy as jnp
from jax import lax
from jax.experimental import pallas as pl
from jax.experimental.pallas import tpu as pltpu
from jax.experimental.pallas import tpu_sc as plsc

_EMB = 64
_CHUNK = 128


@functools.lru_cache(maxsize=None)
def _make_gather(n_rows: int):
    info = plsc.get_sparse_core_info()
    nc, ns = info.num_cores, info.num_subcores
    nw = nc * ns
    assert n_rows % (nw * _CHUNK) == 0
    bpw = n_rows // nw
    nsteps = bpw // _CHUNK

    mesh = plsc.VectorSubcoreMesh(core_axis_name="c", subcore_axis_name="s")

    @functools.partial(
        pl.kernel,
        mesh=mesh,
        compiler_params=pltpu.CompilerParams(use_tc_tiling_on_sc=False),
        out_type=jax.ShapeDtypeStruct((n_rows, _EMB), jnp.float32),
        scratch_types=[
            pltpu.VMEM((nsteps, _CHUNK), jnp.int32),
            pltpu.VMEM((_CHUNK, _EMB), jnp.float32),
            pltpu.SemaphoreType.DMA,
        ],
    )
    def gather(table_hbm, idx_hbm, out_hbm, idx_v, rows_v, sem):
        wid = lax.axis_index("s") * nc + lax.axis_index("c")
        base = wid * bpw
        pltpu.sync_copy(idx_hbm.at[wid], idx_v)

        def step(j, carry):
            pltpu.async_copy(table_hbm.at[idx_v.at[j]], rows_v, sem).wait()
            pltpu.sync_copy(rows_v, out_hbm.at[pl.ds(base + j * _CHUNK, _CHUNK)])
            return carry

        lax.fori_loop(0, nsteps, step, 0)

    def run(table, idx_flat):
        info_nw = nw
        idx3 = idx_flat.reshape(info_nw, nsteps, _CHUNK)
        return gather(table, idx3)

    return run


def kernel(input, lang, table_0, table_1):
    n_rows = input.size
    run = _make_gather(n_rows)
    idx_flat = input.reshape(-1)
    sel = lang.reshape(-1)[0]
    out = lax.cond(
        sel == 0,
        lambda: run(table_0, idx_flat),
        lambda: run(table_1, idx_flat),
    )
    return out.reshape(input.shape + (_EMB,))


# speedup vs baseline: 4.5011x; 1.1169x over previous
import functools

import jax
import jax.numpy as jnp
from jax import lax
from jax.experimental import pallas as pl
from jax.experimental.pallas import tpu as pltpu
from jax.experimental.pallas import tpu_sc as plsc

_EMB = 64
_CHUNK = 128


@functools.lru_cache(maxsize=None)
def _make_gather(n_rows: int):
    info = plsc.get_sparse_core_info()
    nc, ns = info.num_cores, info.num_subcores
    nw = nc * ns
    assert n_rows % (nw * _CHUNK) == 0
    bpw = n_rows // nw
    nsteps = bpw // _CHUNK

    mesh = plsc.VectorSubcoreMesh(core_axis_name="c", subcore_axis_name="s")

    grp = 5
    assert nsteps % grp == 0
    ngroups = nsteps // grp
    grows = grp * _CHUNK

    @functools.partial(
        pl.kernel,
        mesh=mesh,
        compiler_params=pltpu.CompilerParams(use_tc_tiling_on_sc=False),
        out_type=jax.ShapeDtypeStruct((n_rows, _EMB), jnp.float32),
        scratch_types=[
            pltpu.VMEM((nsteps, _CHUNK), jnp.int32),
            pltpu.VMEM((2, grows, _EMB), jnp.float32),
            pltpu.SemaphoreType.DMA,
            pltpu.SemaphoreType.DMA,
            pltpu.SemaphoreType.DMA,
        ],
    )
    def gather(table_hbm, idx_hbm, out_hbm, idx_v, rows_v, gsem, ssem0, ssem1):
        wid = lax.axis_index("s") * nc + lax.axis_index("c")
        base = wid * bpw
        pltpu.sync_copy(idx_hbm.at[wid], idx_v)

        ssems = (ssem0, ssem1)
        pending_store = [None, None]
        for g in range(ngroups):
            b = g % 2
            if pending_store[b] is not None:
                pending_store[b].wait()
            fired = [
                pltpu.async_copy(
                    table_hbm.at[idx_v.at[g * grp + i]],
                    rows_v.at[b, pl.ds(i * _CHUNK, _CHUNK)],
                    gsem,
                )
                for i in range(grp)
            ]
            for c in fired:
                c.wait()
            pending_store[b] = pltpu.async_copy(
                rows_v.at[b],
                out_hbm.at[pl.ds(base + g * grows, grows)],
                ssems[b],
            )
        for c in pending_store:
            if c is not None:
                c.wait()

    def run(table, idx_flat):
        info_nw = nw
        idx3 = idx_flat.reshape(info_nw, nsteps, _CHUNK)
        return gather(table, idx3)

    return run


def kernel(input, lang, table_0, table_1):
    n_rows = input.size
    run = _make_gather(n_rows)
    idx_flat = input.reshape(-1)
    sel = lang.reshape(-1)[0]
    out = lax.cond(
        sel == 0,
        lambda: run(table_0, idx_flat),
        lambda: run(table_1, idx_flat),
    )
    return out.reshape(input.shape + (_EMB,))


# speedup vs baseline: 4.5301x; 1.0064x over previous
import functools

import jax
import jax.numpy as jnp
from jax import lax
from jax.experimental import pallas as pl
from jax.experimental.pallas import tpu as pltpu
from jax.experimental.pallas import tpu_sc as plsc

_EMB = 64
_CHUNK = 128


@functools.lru_cache(maxsize=None)
def _make_gather(n_rows: int):
    info = plsc.get_sparse_core_info()
    nc, ns = info.num_cores, info.num_subcores
    nw = nc * ns
    assert n_rows % (nw * _CHUNK) == 0
    bpw = n_rows // nw
    nsteps = bpw // _CHUNK

    mesh = plsc.VectorSubcoreMesh(core_axis_name="c", subcore_axis_name="s")

    grp = 5
    assert nsteps % grp == 0
    ngroups = nsteps // grp
    grows = grp * _CHUNK

    @functools.partial(
        pl.kernel,
        mesh=mesh,
        compiler_params=pltpu.CompilerParams(use_tc_tiling_on_sc=False),
        out_type=jax.ShapeDtypeStruct((n_rows, _EMB), jnp.float32),
        scratch_types=[
            pltpu.VMEM((nsteps * _CHUNK,), jnp.int32),
            pltpu.VMEM((2, grows, _EMB), jnp.float32),
            pltpu.SemaphoreType.DMA,
            pltpu.SemaphoreType.DMA,
            pltpu.SemaphoreType.DMA,
        ],
    )
    def gather(table_hbm, idx_hbm, out_hbm, idx_v, rows_v, gsem, ssem0, ssem1):
        wid = lax.axis_index("s") * nc + lax.axis_index("c")
        base = wid * bpw
        pltpu.sync_copy(idx_hbm.at[wid], idx_v)

        ssems = (ssem0, ssem1)
        pending_store = [None, None]
        for g in range(ngroups):
            b = g % 2
            if pending_store[b] is not None:
                pending_store[b].wait()
            pltpu.async_copy(
                table_hbm.at[idx_v.at[pl.ds(g * grows, grows)]],
                rows_v.at[b],
                gsem,
            ).wait()
            pending_store[b] = pltpu.async_copy(
                rows_v.at[b],
                out_hbm.at[pl.ds(base + g * grows, grows)],
                ssems[b],
            )
        for c in pending_store:
            if c is not None:
                c.wait()

    def run(table, idx_flat):
        info_nw = nw
        idx3 = idx_flat.reshape(info_nw, nsteps * _CHUNK)
        return gather(table, idx3)

    return run


def kernel(input, lang, table_0, table_1):
    n_rows = input.size
    run = _make_gather(n_rows)
    idx_flat = input.reshape(-1)
    sel = lang.reshape(-1)[0]
    out = lax.cond(
        sel == 0,
        lambda: run(table_0, idx_flat),
        lambda: run(table_1, idx_flat),
    )
    return out.reshape(input.shape + (_EMB,))
